# baseline (speedup 1.0000x reference)
import functools

import jax
import jax.numpy as jnp
from jax import lax
from jax.experimental import pallas as pl
from jax.experimental.pallas import tpu as pltpu
from jax.experimental.pallas import tpu_sc as plsc

_VOCAB = 1000000
_DIM = 64
_BATCH = 16384
_HIST = 200
_NW = 32
_PARTS = 4
_PB = _BATCH // _PARTS
_BPW = _PB // _NW
_NB = 4
_SPLITS = ((0, 104), (104, 96))
_NITER = _BPW // (2 * _NB)

_mesh = plsc.VectorSubcoreMesh(core_axis_name="c", subcore_axis_name="s")


@functools.partial(
    pl.kernel,
    mesh=_mesh,
    out_type=jax.ShapeDtypeStruct((_PB, _HIST, 2 * _DIM), jnp.float32),
    scratch_types=[
        pltpu.VMEM((2 * _NB, _HIST), jnp.int32),
        pltpu.VMEM((_NB, _HIST, _DIM), jnp.float32),
        pltpu.VMEM((_NB, _HIST, _DIM), jnp.float32),
        pltpu.SemaphoreType.DMA,
        pltpu.SemaphoreType.DMA,
        pltpu.SemaphoreType.DMA,
        pltpu.SemaphoreType.DMA,
    ],
    compiler_params=pltpu.CompilerParams(use_tc_tiling_on_sc=False),
)
def _embed_gather(x_hbm, table_hbm, out_hbm, idx_v, rows_a, rows_b,
                  sem_ga, sem_gb, sem_wa, sem_wb):
    wid = lax.axis_index("s") * 2 + lax.axis_index("c")
    b_base = wid * _BPW

    def fire_gathers(rows_buf, j0, sem):
        copies = []
        for j in range(_NB):
            for h, w in _SPLITS:
                copies.append(
                    pltpu.async_copy(
                        table_hbm.at[idx_v.at[j0 + j, pl.ds(h, w)]],
                        rows_buf.at[j, pl.ds(h, w)],
                        sem,
                    )
                )
        return copies

    def out_slice(b0):
        return out_hbm.at[pl.ds(b0, _NB), :, pl.ds(0, _DIM)]

    def body(g, carry):
        b0 = b_base + g * 2 * _NB
        pltpu.sync_copy(x_hbm.at[pl.ds(b0, 2 * _NB)], idx_v)

        out_a = out_slice(b0)
        out_b = out_slice(b0 + _NB)

        @pl.when(g > 0)
        def _():
            pltpu.make_async_copy(rows_a, out_a, sem_wa).wait()

        ga = fire_gathers(rows_a, 0, sem_ga)

        @pl.when(g > 0)
        def _():
            pltpu.make_async_copy(rows_b, out_b, sem_wb).wait()

        gb = fire_gathers(rows_b, _NB, sem_gb)

        for c in ga:
            c.wait()
        pltpu.async_copy(rows_a, out_a, sem_wa)
        for c in gb:
            c.wait()
        pltpu.async_copy(rows_b, out_b, sem_wb)
        return carry

    lax.fori_loop(0, _NITER, body, 0)

    last = b_base + (_NITER - 1) * 2 * _NB
    pltpu.make_async_copy(rows_a, out_slice(last), sem_wa).wait()
    pltpu.make_async_copy(rows_b, out_slice(last + _NB), sem_wb).wait()


def kernel(x, table):
    scale = table[0, 0] * 0.0 + 1.0
    parts = []
    for k in range(_PARTS):
        padded = _embed_gather(x[k * _PB:(k + 1) * _PB], table)
        parts.append(padded[:, :, :_DIM] * scale)
    return jnp.concatenate(parts, axis=0)


# speedup vs baseline: 1.8936x; 1.8936x over previous
import functools

import jax
import jax.numpy as jnp
from jax import lax
from jax.experimental import pallas as pl
from jax.experimental.pallas import tpu as pltpu
from jax.experimental.pallas import tpu_sc as plsc

_VOCAB = 1000000
_DIM = 64
_BATCH = 16384
_HIST = 200
_NW = 32
_BPW = _BATCH // _NW
_NB = 4
_SPLITS = ((0, 104), (104, 96))
_NITER = _BPW // (2 * _NB)

_mesh = plsc.VectorSubcoreMesh(core_axis_name="c", subcore_axis_name="s")


@functools.partial(
    pl.kernel,
    mesh=_mesh,
    out_type=jax.ShapeDtypeStruct((_BATCH, _HIST, 2 * _DIM), jnp.float32),
    scratch_types=[
        pltpu.VMEM((2 * _NB, _HIST), jnp.int32),
        pltpu.VMEM((_NB, _HIST, _DIM), jnp.float32),
        pltpu.VMEM((_NB, _HIST, _DIM), jnp.float32),
        pltpu.SemaphoreType.DMA,
        pltpu.SemaphoreType.DMA,
        pltpu.SemaphoreType.DMA,
        pltpu.SemaphoreType.DMA,
    ],
    compiler_params=pltpu.CompilerParams(use_tc_tiling_on_sc=False),
)
def _embed_gather(x_hbm, table_hbm, out_hbm, idx_v, rows_a, rows_b,
                  sem_ga, sem_gb, sem_wa, sem_wb):
    wid = lax.axis_index("s") * 2 + lax.axis_index("c")
    b_base = wid * _BPW

    def fire_gathers(rows_buf, j0, sem):
        copies = []
        for j in range(_NB):
            for h, w in _SPLITS:
                copies.append(
                    pltpu.async_copy(
                        table_hbm.at[idx_v.at[j0 + j, pl.ds(h, w)]],
                        rows_buf.at[j, pl.ds(h, w)],
                        sem,
                    )
                )
        return copies

    def out_slice(b0):
        return out_hbm.at[pl.ds(b0, _NB), :, pl.ds(0, _DIM)]

    def body(g, carry):
        b0 = b_base + g * 2 * _NB
        pltpu.sync_copy(x_hbm.at[pl.ds(b0, 2 * _NB)], idx_v)

        out_a = out_slice(b0)
        out_b = out_slice(b0 + _NB)

        @pl.when(g > 0)
        def _():
            pltpu.make_async_copy(rows_a, out_a, sem_wa).wait()

        ga = fire_gathers(rows_a, 0, sem_ga)

        @pl.when(g > 0)
        def _():
            pltpu.make_async_copy(rows_b, out_b, sem_wb).wait()

        gb = fire_gathers(rows_b, _NB, sem_gb)

        for c in ga:
            c.wait()
        pltpu.async_copy(rows_a, out_a, sem_wa)
        for c in gb:
            c.wait()
        pltpu.async_copy(rows_b, out_b, sem_wb)
        return carry

    lax.fori_loop(0, _NITER, body, 0)

    last = b_base + (_NITER - 1) * 2 * _NB
    pltpu.make_async_copy(rows_a, out_slice(last), sem_wa).wait()
    pltpu.make_async_copy(rows_b, out_slice(last + _NB), sem_wb).wait()


def kernel(x, table):
    scale = table[0, 0] * 0.0 + 1.0
    table_lin = table * scale
    padded = _embed_gather(x, table_lin)
    return padded[:, :, :_DIM]


# speedup vs baseline: 2.2135x; 1.1689x over previous
import functools

import jax
import jax.numpy as jnp
from jax import lax
from jax.experimental import pallas as pl
from jax.experimental.pallas import tpu as pltpu
from jax.experimental.pallas import tpu_sc as plsc

_VOCAB = 1000000
_DIM = 64
_BATCH = 16384
_HIST = 200
_NW = 32
_BPW = _BATCH // _NW
_NB = 4
_SPLITS = ((0, 104), (104, 96))
_NITER = _BPW // (2 * _NB)

_mesh = plsc.VectorSubcoreMesh(core_axis_name="c", subcore_axis_name="s")


@functools.partial(
    pl.kernel,
    mesh=_mesh,
    out_type=jax.ShapeDtypeStruct((_BATCH, _HIST, 2 * _DIM), jnp.float32),
    scratch_types=[
        pltpu.VMEM((2 * _NB, _HIST), jnp.int32),
        pltpu.VMEM((_NB, _HIST, _DIM), jnp.float32),
        pltpu.VMEM((_NB, _HIST, _DIM), jnp.float32),
        pltpu.SemaphoreType.DMA,
        pltpu.SemaphoreType.DMA,
        pltpu.SemaphoreType.DMA,
        pltpu.SemaphoreType.DMA,
    ],
    compiler_params=pltpu.CompilerParams(use_tc_tiling_on_sc=False),
)
def _embed_gather(x_hbm, table_hbm, out_hbm, idx_v, rows_a, rows_b,
                  sem_ga, sem_gb, sem_wa, sem_wb):
    wid = lax.axis_index("s") * 2 + lax.axis_index("c")
    b_base = wid * _BPW

    def fire_gathers(rows_buf, j0, sem):
        copies = []
        for j in range(_NB):
            for h, w in _SPLITS:
                copies.append(
                    pltpu.async_copy(
                        table_hbm.at[idx_v.at[j0 + j, pl.ds(h, w)]],
                        rows_buf.at[j, pl.ds(h, w)],
                        sem,
                    )
                )
        return copies

    def out_slice(b0):
        return out_hbm.at[pl.ds(b0, _NB), :, pl.ds(0, _DIM)]

    def body(g, carry):
        b0 = b_base + g * 2 * _NB
        pltpu.sync_copy(x_hbm.at[pl.ds(b0, 2 * _NB)], idx_v)

        out_a = out_slice(b0)
        out_b = out_slice(b0 + _NB)

        @pl.when(g > 0)
        def _():
            pltpu.make_async_copy(rows_a, out_a, sem_wa).wait()

        ga = fire_gathers(rows_a, 0, sem_ga)

        @pl.when(g > 0)
        def _():
            pltpu.make_async_copy(rows_b, out_b, sem_wb).wait()

        gb = fire_gathers(rows_b, _NB, sem_gb)

        for c in ga:
            c.wait()
        pltpu.async_copy(rows_a, out_a, sem_wa)
        for c in gb:
            c.wait()
        pltpu.async_copy(rows_b, out_b, sem_wb)
        return carry

    lax.fori_loop(0, _NITER, body, 0)

    last = b_base + (_NITER - 1) * 2 * _NB
    pltpu.make_async_copy(rows_a, out_slice(last), sem_wa).wait()
    pltpu.make_async_copy(rows_b, out_slice(last + _NB), sem_wb).wait()


def kernel(x, table):
    padded = _embed_gather(x, table)
    return padded[:, :, :_DIM]


# speedup vs baseline: 2.2209x; 1.0034x over previous
import functools

import jax
import jax.numpy as jnp
from jax import lax
from jax.experimental import pallas as pl
from jax.experimental.pallas import tpu as pltpu
from jax.experimental.pallas import tpu_sc as plsc

_VOCAB = 1000000
_DIM = 64
_BATCH = 16384
_HIST = 200
_NW = 32
_BPW = _BATCH // _NW
_NB = 4
_SPLITS = ((0, 104), (104, 96))
_GRP = 4 * _NB
_NITER = _BPW // _GRP

_mesh = plsc.VectorSubcoreMesh(core_axis_name="c", subcore_axis_name="s")


@functools.partial(
    pl.kernel,
    mesh=_mesh,
    out_type=jax.ShapeDtypeStruct((_BATCH, _HIST, 2 * _DIM), jnp.float32),
    scratch_types=[
        pltpu.VMEM((2 * _NB, _HIST), jnp.int32),
        pltpu.VMEM((2 * _NB, _HIST), jnp.int32),
        pltpu.VMEM((_NB, _HIST, _DIM), jnp.float32),
        pltpu.VMEM((_NB, _HIST, _DIM), jnp.float32),
        pltpu.SemaphoreType.DMA,
        pltpu.SemaphoreType.DMA,
        pltpu.SemaphoreType.DMA,
        pltpu.SemaphoreType.DMA,
        pltpu.SemaphoreType.DMA,
        pltpu.SemaphoreType.DMA,
    ],
    compiler_params=pltpu.CompilerParams(use_tc_tiling_on_sc=False),
)
def _embed_gather(x_hbm, table_hbm, out_hbm, idx_v, idx_w, rows_a, rows_b,
                  sem_ga, sem_gb, sem_wa, sem_wb, sem_iv, sem_iw):
    wid = lax.axis_index("s") * 2 + lax.axis_index("c")
    b_base = wid * _BPW

    def fire_gathers(idx_buf, rows_buf, j0, sem):
        copies = []
        for j in range(_NB):
            for h, w in _SPLITS:
                copies.append(
                    pltpu.async_copy(
                        table_hbm.at[idx_buf.at[j0 + j, pl.ds(h, w)]],
                        rows_buf.at[j, pl.ds(h, w)],
                        sem,
                    )
                )
        return copies

    def out_slice(b0):
        return out_hbm.at[pl.ds(b0, _NB), :, pl.ds(0, _DIM)]

    pltpu.sync_copy(x_hbm.at[pl.ds(b_base, 2 * _NB)], idx_v)

    def body(g, carry):
        b0 = b_base + g * _GRP

        pltpu.async_copy(x_hbm.at[pl.ds(b0 + 2 * _NB, 2 * _NB)], idx_w,
                         sem_iw)

        @pl.when(g > 0)
        def _():
            pltpu.make_async_copy(
                x_hbm.at[pl.ds(b0, 2 * _NB)], idx_v, sem_iv).wait()

        out_c0 = out_slice(b0)
        out_c1 = out_slice(b0 + _NB)
        out_c2 = out_slice(b0 + 2 * _NB)
        out_c3 = out_slice(b0 + 3 * _NB)

        @pl.when(g > 0)
        def _():
            pltpu.make_async_copy(rows_a, out_c2, sem_wa).wait()

        ga = fire_gathers(idx_v, rows_a, 0, sem_ga)

        @pl.when(g > 0)
        def _():
            pltpu.make_async_copy(rows_b, out_c3, sem_wb).wait()

        gb = fire_gathers(idx_v, rows_b, _NB, sem_gb)

        for c in ga:
            c.wait()
        pltpu.async_copy(rows_a, out_c0, sem_wa)
        for c in gb:
            c.wait()
        pltpu.async_copy(rows_b, out_c1, sem_wb)

        pltpu.make_async_copy(
            x_hbm.at[pl.ds(b0 + 2 * _NB, 2 * _NB)], idx_w, sem_iw).wait()

        pltpu.make_async_copy(rows_a, out_c0, sem_wa).wait()
        ga2 = fire_gathers(idx_w, rows_a, 0, sem_ga)

        pltpu.make_async_copy(rows_b, out_c1, sem_wb).wait()
        gb2 = fire_gathers(idx_w, rows_b, _NB, sem_gb)

        @pl.when(g < _NITER - 1)
        def _():
            pltpu.async_copy(
                x_hbm.at[pl.ds(b0 + _GRP, 2 * _NB)], idx_v, sem_iv)

        for c in ga2:
            c.wait()
        pltpu.async_copy(rows_a, out_c2, sem_wa)
        for c in gb2:
            c.wait()
        pltpu.async_copy(rows_b, out_c3, sem_wb)
        return carry

    lax.fori_loop(0, _NITER, body, 0)

    last = b_base + (_NITER - 1) * _GRP
    pltpu.make_async_copy(rows_a, out_slice(last + 2 * _NB), sem_wa).wait()
    pltpu.make_async_copy(rows_b, out_slice(last + 3 * _NB), sem_wb).wait()


def kernel(x, table):
    padded = _embed_gather(x, table)
    return padded[:, :, :_DIM]
